# baseline (speedup 1.0000x reference)
import functools

import jax
import jax.numpy as jnp
from jax.experimental import pallas as pl
from jax.experimental.pallas import tpu as pltpu

_LN_EPS = 1e-5


def _ffn_body(x_ref, w1_ref, b1_ref, w2_ref, pdb_ref, seg_ref, o_ref, *, inv_d):
    x = x_ref[...]
    if x.dtype != jnp.float32:
        x = x.astype(jnp.float32)

    xb = x.astype(jnp.bfloat16)
    h = jnp.dot(xb, w1_ref[...], preferred_element_type=jnp.float32)
    h = jnp.maximum(h + b1_ref[...], 0.0).astype(jnp.bfloat16)
    y = jnp.dot(h, w2_ref[...], preferred_element_type=jnp.float32)

    z = y + pdb_ref[0:1, :] + x
    seg = seg_ref[...]
    mean = jnp.dot(z.astype(jnp.bfloat16), seg,
                   preferred_element_type=jnp.float32) * inv_d
    d_c = z - mean
    var = jnp.dot((d_c * d_c).astype(jnp.bfloat16), seg,
                  preferred_element_type=jnp.float32) * inv_d
    zn = d_c * jax.lax.rsqrt(var + _LN_EPS)
    o_ref[...] = (zn * pdb_ref[1:2, :] + pdb_ref[2:3, :]).astype(o_ref.dtype)


def _cnn_ffn(x, w1t, b1, w2t, b2, gamma, beta, *, tile_rows=2048):
    seq, batch, d = x.shape
    f = w1t.shape[1]
    n = seq * batch
    dtype = x.dtype

    pack = 128 // d if (d <= 128 and 128 % d == 0) else 1
    while pack > 1 and n % pack != 0:
        pack //= 2
    pack = max(pack, 1)

    dp, fp = pack * d, pack * f
    m = n // pack
    x2d = x.reshape(m, dp)

    eye = jnp.eye(pack, dtype=jnp.float32)
    w1e = jnp.kron(eye, w1t.astype(jnp.float32)).astype(jnp.bfloat16)
    w2e = jnp.kron(eye, w2t.astype(jnp.float32)).astype(jnp.bfloat16)
    seg = jnp.kron(eye, jnp.ones((d, d), jnp.float32)).astype(jnp.bfloat16)
    b1e = jnp.tile(b1.reshape(1, f).astype(jnp.float32), (1, pack))
    pdb = jnp.concatenate(
        [jnp.tile(v.reshape(1, d).astype(jnp.float32), (1, pack))
         for v in (b2, gamma, beta)], axis=0)

    tile_rows = max(8, min(tile_rows, m))
    grid = (pl.cdiv(m, tile_rows),)

    flops = 2 * m * dp * fp + 2 * m * fp * dp + 2 * 2 * m * dp * dp
    bytes_accessed = 4 * (2 * m * dp) + 2 * (dp * fp + fp * dp + dp * dp)
    cost = pl.CostEstimate(flops=int(flops), transcendentals=int(m * dp),
                           bytes_accessed=int(bytes_accessed))

    out2d = pl.pallas_call(
        functools.partial(_ffn_body, inv_d=1.0 / d),
        out_shape=jax.ShapeDtypeStruct((m, dp), dtype),
        grid_spec=pltpu.PrefetchScalarGridSpec(
            num_scalar_prefetch=0,
            grid=grid,
            in_specs=[
                pl.BlockSpec((tile_rows, dp), lambda i: (i, 0)),
                pl.BlockSpec((dp, fp), lambda i: (0, 0)),
                pl.BlockSpec((1, fp), lambda i: (0, 0)),
                pl.BlockSpec((fp, dp), lambda i: (0, 0)),
                pl.BlockSpec((3, dp), lambda i: (0, 0)),
                pl.BlockSpec((dp, dp), lambda i: (0, 0)),
            ],
            out_specs=pl.BlockSpec((tile_rows, dp), lambda i: (i, 0)),
        ),
        compiler_params=pltpu.CompilerParams(
            dimension_semantics=("parallel",)),
        cost_estimate=cost,
    )(x2d, w1e, b1e, w2e, pdb, seg)

    return out2d.reshape(seq, batch, d)


def kernel(x, w1t, b1, w2t, b2, gamma, beta):
    return _cnn_ffn(x, w1t, b1, w2t, b2, gamma, beta)


# speedup vs baseline: 1.1736x; 1.1736x over previous
import functools

import jax
import jax.numpy as jnp
from jax.experimental import pallas as pl
from jax.experimental.pallas import tpu as pltpu

_LN_EPS = 1e-5


def _ffn_body(x_ref, w1_ref, b1_ref, w2_ref, b2_ref, g_ref, bt_ref, o_ref,
              *, inv_d):
    s_blk, batch, d = x_ref.shape
    rows = s_blk * batch

    x = x_ref[...].reshape(rows, d)
    if x.dtype != jnp.float32:
        x = x.astype(jnp.float32)

    w1 = w1_ref[...].astype(jnp.bfloat16)
    w2 = w2_ref[...].astype(jnp.bfloat16)

    h = jnp.dot(x.astype(jnp.bfloat16), w1,
                preferred_element_type=jnp.float32)
    h = jnp.maximum(h + b1_ref[...], 0.0).astype(jnp.bfloat16)
    y = jnp.dot(h, w2, preferred_element_type=jnp.float32)

    z = y + b2_ref[...] + x
    mean = jnp.sum(z, axis=-1, keepdims=True) * inv_d
    d_c = z - mean
    var = jnp.sum(d_c * d_c, axis=-1, keepdims=True) * inv_d
    zn = d_c * jax.lax.rsqrt(var + _LN_EPS)
    o = zn * g_ref[...] + bt_ref[...]
    o_ref[...] = o.reshape(s_blk, batch, d).astype(o_ref.dtype)


def kernel(x, w1t, b1, w2t, b2, gamma, beta):
    seq, batch, d = x.shape
    f = w1t.shape[1]
    dtype = x.dtype

    b1r = b1.reshape(1, f)
    b2r = b2.reshape(1, d)
    gr = gamma.reshape(1, d)
    btr = beta.reshape(1, d)

    s_blk = min(8, seq)
    grid = (pl.cdiv(seq, s_blk),)

    n = seq * batch
    flops = 2 * n * d * f * 2 + 8 * n * d
    bytes_accessed = 4 * (2 * n * d + 2 * d * f + f + 3 * d)
    cost = pl.CostEstimate(flops=int(flops), transcendentals=int(n),
                           bytes_accessed=int(bytes_accessed))

    out = pl.pallas_call(
        functools.partial(_ffn_body, inv_d=1.0 / d),
        out_shape=jax.ShapeDtypeStruct((seq, batch, d), dtype),
        grid_spec=pltpu.PrefetchScalarGridSpec(
            num_scalar_prefetch=0,
            grid=grid,
            in_specs=[
                pl.BlockSpec((s_blk, batch, d), lambda i: (i, 0, 0)),
                pl.BlockSpec((d, f), lambda i: (0, 0)),
                pl.BlockSpec((1, f), lambda i: (0, 0)),
                pl.BlockSpec((f, d), lambda i: (0, 0)),
                pl.BlockSpec((1, d), lambda i: (0, 0)),
                pl.BlockSpec((1, d), lambda i: (0, 0)),
                pl.BlockSpec((1, d), lambda i: (0, 0)),
            ],
            out_specs=pl.BlockSpec((s_blk, batch, d), lambda i: (i, 0, 0)),
        ),
        compiler_params=pltpu.CompilerParams(
            dimension_semantics=("parallel",)),
        cost_estimate=cost,
    )(x, w1t, b1r, w2t, b2r, gr, btr)

    return out


# speedup vs baseline: 4.1452x; 3.5320x over previous
import functools

import jax
import jax.numpy as jnp
from jax.experimental import pallas as pl
from jax.experimental.pallas import tpu as pltpu

_LN_EPS = 1e-5


def _ffn_body(x_ref, w1_ref, b1_ref, w2_ref, b2_ref, g_ref, bt_ref, o_ref,
              *, inv_d):
    s_blk = x_ref.shape[0]
    w1 = w1_ref[...].astype(jnp.bfloat16)
    w2 = w2_ref[...].astype(jnp.bfloat16)
    b1 = b1_ref[...]
    b2 = b2_ref[...]
    g = g_ref[...]
    bt = bt_ref[...]

    for s in range(s_blk):
        x = x_ref[s]
        h = jnp.dot(w1, x.astype(jnp.bfloat16),
                    preferred_element_type=jnp.float32)
        h = jnp.maximum(h + b1, 0.0).astype(jnp.bfloat16)
        y = jnp.dot(w2, h, preferred_element_type=jnp.float32)
        z = y + b2 + x
        mean = jnp.sum(z, axis=0, keepdims=True) * inv_d
        d_c = z - mean
        var = jnp.sum(d_c * d_c, axis=0, keepdims=True) * inv_d
        zn = d_c * jax.lax.rsqrt(var + _LN_EPS)
        o_ref[s] = (zn * g + bt).astype(o_ref.dtype)


def kernel(x, w1t, b1, w2t, b2, gamma, beta):
    seq, batch, d = x.shape
    f = w1t.shape[1]
    dtype = x.dtype

    xt = jnp.transpose(x, (0, 2, 1))
    w1 = w1t.T
    w2 = w2t.T
    b1r = b1.reshape(f, 1)
    b2r = b2.reshape(d, 1)
    gr = gamma.reshape(d, 1)
    btr = beta.reshape(d, 1)

    s_blk = min(8, seq)
    grid = (pl.cdiv(seq, s_blk),)

    n = seq * batch
    flops = 2 * n * d * f * 2 + 8 * n * d
    bytes_accessed = 4 * (2 * n * d + 2 * d * f + f + 3 * d)
    cost = pl.CostEstimate(flops=int(flops), transcendentals=int(n),
                           bytes_accessed=int(bytes_accessed))

    out_t = pl.pallas_call(
        functools.partial(_ffn_body, inv_d=1.0 / d),
        out_shape=jax.ShapeDtypeStruct((seq, d, batch), dtype),
        grid_spec=pltpu.PrefetchScalarGridSpec(
            num_scalar_prefetch=0,
            grid=grid,
            in_specs=[
                pl.BlockSpec((s_blk, d, batch), lambda i: (i, 0, 0)),
                pl.BlockSpec((f, d), lambda i: (0, 0)),
                pl.BlockSpec((f, 1), lambda i: (0, 0)),
                pl.BlockSpec((d, f), lambda i: (0, 0)),
                pl.BlockSpec((d, 1), lambda i: (0, 0)),
                pl.BlockSpec((d, 1), lambda i: (0, 0)),
                pl.BlockSpec((d, 1), lambda i: (0, 0)),
            ],
            out_specs=pl.BlockSpec((s_blk, d, batch), lambda i: (i, 0, 0)),
        ),
        compiler_params=pltpu.CompilerParams(
            dimension_semantics=("parallel",)),
        cost_estimate=cost,
    )(xt, w1, b1r, w2, b2r, gr, btr)

    return jnp.transpose(out_t, (0, 2, 1))
